# baseline (speedup 1.0000x reference)
import functools

import jax
import jax.numpy as jnp
from jax import lax
from jax.experimental import pallas as pl
from jax.experimental.pallas import tpu as pltpu
from jax.experimental.pallas import tpu_sc as plsc

D = 64
NC, NS = 2, 16
NW = NC * NS
ROWS_PER_GATHER = 128
K = 4
CHUNK = K * ROWS_PER_GATHER


@functools.partial(jax.jit, static_argnums=(2,))
def _gather(table, idx2d, n_rows):
    per_w = n_rows // NW
    n_chunk = per_w // CHUNK
    idx_rows_per_w = per_w // ROWS_PER_GATHER

    mesh = plsc.VectorSubcoreMesh(
        core_axis_name="c", subcore_axis_name="s",
        num_cores=NC, num_subcores=NS)

    @functools.partial(
        pl.kernel,
        mesh=mesh,
        compiler_params=pltpu.CompilerParams(use_tc_tiling_on_sc=False),
        out_type=jax.ShapeDtypeStruct((n_rows, D), jnp.float32),
        scratch_types=[
            pltpu.VMEM((K, ROWS_PER_GATHER), jnp.int32),
            pltpu.VMEM((CHUNK, D), jnp.float32),
            pltpu.SemaphoreType.DMA,
        ],
    )
    def body(table_hbm, idx_hbm, out_hbm, idx_v, rows_v, sem):
        wid = lax.axis_index("s") * NC + lax.axis_index("c")
        idx_row0 = wid * idx_rows_per_w
        out_row0 = wid * per_w

        @pl.loop(0, n_chunk)
        def _(g):
            pltpu.sync_copy(idx_hbm.at[pl.ds(idx_row0 + g * K, K)], idx_v)
            copies = [
                pltpu.async_copy(
                    table_hbm.at[idx_v.at[j]],
                    rows_v.at[pl.ds(j * ROWS_PER_GATHER, ROWS_PER_GATHER)],
                    sem)
                for j in range(K)
            ]
            for cp in copies:
                cp.wait()
            pltpu.sync_copy(rows_v, out_hbm.at[pl.ds(out_row0 + g * CHUNK, CHUNK)])

    return body(table, idx2d)


def kernel(x, embedding_table, train):
    b, s = x.shape
    n = b * s
    idx2d = x.reshape(n // ROWS_PER_GATHER, ROWS_PER_GATHER)
    out = _gather(embedding_table, idx2d, n)
    return out.reshape(b, s, D)


# speedup vs baseline: 1.0455x; 1.0455x over previous
import functools

import jax
import jax.numpy as jnp
from jax import lax
from jax.experimental import pallas as pl
from jax.experimental.pallas import tpu as pltpu
from jax.experimental.pallas import tpu_sc as plsc

D = 64
NC, NS = 2, 16
NW = NC * NS
ROWS_PER_GATHER = 128
K = 4
CHUNK = K * ROWS_PER_GATHER
NBUF = 2


@functools.partial(jax.jit, static_argnums=(2,))
def _gather(table, idx2d, n_rows):
    per_w = n_rows // NW
    n_chunk = per_w // CHUNK
    idx_rows_per_w = per_w // ROWS_PER_GATHER

    mesh = plsc.VectorSubcoreMesh(
        core_axis_name="c", subcore_axis_name="s",
        num_cores=NC, num_subcores=NS)

    @functools.partial(
        pl.kernel,
        mesh=mesh,
        compiler_params=pltpu.CompilerParams(use_tc_tiling_on_sc=False),
        out_type=jax.ShapeDtypeStruct((n_rows, D), jnp.float32),
        scratch_types=[
            pltpu.VMEM((idx_rows_per_w, ROWS_PER_GATHER), jnp.int32),
            pltpu.VMEM((NBUF, CHUNK, D), jnp.float32),
            pltpu.SemaphoreType.DMA,
        ],
    )
    def body(table_hbm, idx_hbm, out_hbm, idx_v, rows_v, sem):
        wid = lax.axis_index("s") * NC + lax.axis_index("c")
        idx_row0 = wid * idx_rows_per_w
        out_row0 = wid * per_w

        pltpu.sync_copy(idx_hbm.at[pl.ds(idx_row0, idx_rows_per_w)], idx_v)

        def fire(g, b):
            for j in range(K):
                pltpu.async_copy(
                    table_hbm.at[idx_v.at[g * K + j]],
                    rows_v.at[b, pl.ds(j * ROWS_PER_GATHER, ROWS_PER_GATHER)],
                    sem)

        def drain(g, b):
            for j in range(K):
                pltpu.make_async_copy(
                    table_hbm.at[idx_v.at[g * K + j]],
                    rows_v.at[b, pl.ds(j * ROWS_PER_GATHER, ROWS_PER_GATHER)],
                    sem).wait()

        fire(0, 0)

        @pl.loop(0, n_chunk, step=NBUF)
        def _(g2):
            for b in range(NBUF):
                g = g2 + b
                nb = (b + 1) % NBUF

                @pl.when(g + 1 < n_chunk)
                def _():
                    fire(g + 1, nb)

                drain(g, b)
                pltpu.sync_copy(
                    rows_v.at[b],
                    out_hbm.at[pl.ds(out_row0 + g * CHUNK, CHUNK)])

    return body(table, idx2d)


def kernel(x, embedding_table, train):
    b, s = x.shape
    n = b * s
    idx2d = x.reshape(n // ROWS_PER_GATHER, ROWS_PER_GATHER)
    out = _gather(embedding_table, idx2d, n)
    return out.reshape(b, s, D)
